# baseline (speedup 1.0000x reference)
import jax
import jax.numpy as jnp
from jax import lax
from jax.experimental import pallas as pl
from jax.experimental.pallas import tpu as pltpu
from jax.experimental.pallas import tpu_sc as plsc

B = 16384
D = 32
OD = 4 * D
SL = 20
NC = 2
NS = 16
NW = NC * NS
BPW = B // NW
CHUNK = 128
NCHUNK = BPW // CHUNK
NPASS = 2
CPP = NCHUNK // NPASS
RPP = BPW // NPASS
TROWS = 25000
TITLE_OOV = 100000
LOCV = 1001
LEVV = 101
SKV = 51


def _sc_body(item1, loc_i, lev_i, skill_i,
             title_r, oov_h, loctab_h, levtab_h, sktab_h,
             out,
             tio, tidx4, li, vi, si, oov, loctab, levtab, sktab,
             tile, rb0, rb1, sem):
    wid = lax.axis_index("s") * NC + lax.axis_index("c")
    base = wid * BPW

    pltpu.sync_copy(item1.at[pl.ds(base, BPW)], tio)
    pltpu.sync_copy(loc_i.at[pl.ds(base, BPW)], li)
    pltpu.sync_copy(lev_i.at[pl.ds(base, BPW)], vi)
    pltpu.sync_copy(skill_i.at[pl.ds(base * SL, BPW * SL)], si)
    pltpu.sync_copy(oov_h, oov)
    pltpu.sync_copy(loctab_h, loctab)
    pltpu.sync_copy(levtab_h, levtab)
    pltpu.sync_copy(sktab_h, sktab)

    for k in range(BPW // 16):
        v = jnp.minimum(tio[pl.ds(k * 16, 16)] >> 2, TROWS - 1)
        tidx4[k // (CHUNK // 16), pl.ds((k % (CHUNK // 16)) * 16, 16)] = v

    rbufs = [rb0, rb1]
    lane = lax.iota(jnp.int32, 16)
    inv_len = jnp.float32(1.0 / SL)

    def one_pass(p, carry):
        prow0 = p * RPP
        cps = [pltpu.async_copy(title_r.at[tidx4.at[p * CPP + j]], rbufs[j], sem)
               for j in range(CPP)]

        @plsc.parallel_loop(0, RPP // 16, unroll=3)
        def grp(g):
            cols = pl.ds(prow0 + g * 16, 16)
            lv = li[cols]
            vv = vi[cols]
            b_sl = (prow0 + g * 16 + lane) * SL
            toks = [plsc.load_gather(si, [b_sl + l]) for l in range(SL)]
            row_off = (g * 16 + lane) * OD
            for t in range(D // 2):
                dvec = (lane + t) & (D // 2 - 1)
                wl = plsc.bitcast(
                    plsc.load_gather(loctab, [lv + dvec * LOCV]), jnp.bfloat16)
                a, b = plsc.unpack(wl, format=plsc.PackFormat.INTERLEAVED)
                plsc.store_scatter(tile, [row_off + D + dvec], a)
                plsc.store_scatter(tile, [row_off + D + 16 + dvec], b)
                wv = plsc.bitcast(
                    plsc.load_gather(levtab, [vv + dvec * LEVV]), jnp.bfloat16)
                a, b = plsc.unpack(wv, format=plsc.PackFormat.INTERLEAVED)
                plsc.store_scatter(tile, [row_off + 2 * D + dvec], a)
                plsc.store_scatter(tile, [row_off + 2 * D + 16 + dvec], b)
                dsk = dvec * SKV
                vals = [plsc.bitcast(plsc.load_gather(sktab, [toks[l] + dsk]),
                                     jnp.bfloat16)
                        for l in range(SL)]
                while len(vals) > 1:
                    vals = [a + b for a, b in zip(vals[::2], vals[1::2])] + (
                        [vals[-1]] if len(vals) % 2 else [])
                a, b = plsc.unpack(vals[0], format=plsc.PackFormat.INTERLEAVED)
                plsc.store_scatter(tile, [row_off + 3 * D + dvec], a * inv_len)
                plsc.store_scatter(tile, [row_off + 3 * D + 16 + dvec],
                                   b * inv_len)

        for j in range(CPP):
            cps[j].wait()
            rbj = rbufs[j]
            crow0 = prow0 + j * CHUNK

            @plsc.parallel_loop(0, CHUNK // 16, unroll=2)
            def tgrp(k):
                r_vec = k * 16 + lane
                idxv = tio[pl.ds(crow0 + k * 16, 16)]
                is_oov = idxv == TITLE_OOV
                sub = (idxv & 3) * D
                row_off = (j * CHUNK + k * 16 + lane) * OD
                for t in range(D):
                    dvec = (lane + t) & (D - 1)
                    val = plsc.load_gather(rbj, [r_vec, sub + dvec])
                    val = jnp.where(is_oov, plsc.load_gather(oov, [dvec]), val)
                    plsc.store_scatter(tile, [row_off + dvec], val)

        pltpu.sync_copy(tile, out.at[pl.ds((base + prow0) * OD, RPP * OD)])
        return carry

    lax.fori_loop(0, NPASS, one_pass, 0)


@jax.jit
def kernel(item1, location_item1, level_item1, skill_text_item1,
           title_table, location_table, level_table, skill_table):
    mesh = plsc.VectorSubcoreMesh(core_axis_name="c", subcore_axis_name="s",
                                  num_cores=NC, num_subcores=NS)
    f32 = jnp.float32
    title_r = title_table[:4 * TROWS].reshape(TROWS, 4 * D)
    oov_row = title_table[TITLE_OOV]

    def pack_pairs(tab):
        bf = tab.astype(jnp.bfloat16)
        lo = lax.bitcast_convert_type(bf[:, :16], jnp.uint16).astype(jnp.uint32)
        hi = lax.bitcast_convert_type(bf[:, 16:], jnp.uint16).astype(jnp.uint32)
        return (lo | (hi << 16)).astype(jnp.int32).T.reshape(-1)

    loctab_t = pack_pairs(location_table)
    levtab_t = pack_pairs(level_table)
    sktab_t = pack_pairs(skill_table)
    run = pl.kernel(
        _sc_body,
        out_type=jax.ShapeDtypeStruct((B * OD,), f32),
        mesh=mesh,
        compiler_params=pltpu.CompilerParams(needs_layout_passes=False),
        scratch_types=[
            pltpu.VMEM((BPW,), jnp.int32),
            pltpu.VMEM((NCHUNK, CHUNK), jnp.int32),
            pltpu.VMEM((BPW,), jnp.int32),
            pltpu.VMEM((BPW,), jnp.int32),
            pltpu.VMEM((BPW * SL,), jnp.int32),
            pltpu.VMEM((D,), f32),
            pltpu.VMEM((D // 2 * LOCV,), jnp.int32),
            pltpu.VMEM((D // 2 * LEVV,), jnp.int32),
            pltpu.VMEM((D // 2 * SKV,), jnp.int32),
            pltpu.VMEM((RPP * OD,), f32),
            pltpu.VMEM((CHUNK, 4 * D), f32),
            pltpu.VMEM((CHUNK, 4 * D), f32),
            pltpu.SemaphoreType.DMA,
        ],
    )
    flat = run(item1, location_item1, level_item1,
               skill_text_item1.reshape(-1),
               title_r, oov_row, loctab_t, levtab_t, sktab_t)
    return flat.reshape(B, OD)


# speedup vs baseline: 1.0789x; 1.0789x over previous
import jax
import jax.numpy as jnp
from jax import lax
from jax.experimental import pallas as pl
from jax.experimental.pallas import tpu as pltpu
from jax.experimental.pallas import tpu_sc as plsc

B = 16384
D = 32
OD = 4 * D
SL = 20
NC = 2
NS = 16
NW = NC * NS
BPW = B // NW
CHUNK = 128
NCHUNK = BPW // CHUNK
NPASS = 2
CPP = NCHUNK // NPASS
RPP = BPW // NPASS
TROWS = 25000
TITLE_OOV = 100000
LOCV = 1001
LEVV = 101
SKV = 51


def _sc_body(item1, loc_i, lev_i, skill_i,
             title_r, oov_h, loctab_h, levtab_h, sktab_h,
             out,
             tio, tidx4, li, vi, si, oov, loctab, levtab, sktab,
             tile, rb0, rb1, sem):
    wid = lax.axis_index("s") * NC + lax.axis_index("c")
    base = wid * BPW

    pltpu.sync_copy(item1.at[pl.ds(base, BPW)], tio)
    pltpu.sync_copy(loc_i.at[pl.ds(base, BPW)], li)
    pltpu.sync_copy(lev_i.at[pl.ds(base, BPW)], vi)
    pltpu.sync_copy(skill_i.at[pl.ds(base * SL, BPW * SL)], si)
    pltpu.sync_copy(oov_h, oov)
    pltpu.sync_copy(loctab_h, loctab)
    pltpu.sync_copy(levtab_h, levtab)
    pltpu.sync_copy(sktab_h, sktab)

    for k in range(BPW // 16):
        v = jnp.minimum(tio[pl.ds(k * 16, 16)] >> 2, TROWS - 1)
        tidx4[k // (CHUNK // 16), pl.ds((k % (CHUNK // 16)) * 16, 16)] = v

    rbufs = [rb0, rb1]
    lane = lax.iota(jnp.int32, 16)
    inv_len = jnp.float32(1.0 / SL)

    def one_pass(p, carry):
        prow0 = p * RPP
        cps = [pltpu.async_copy(title_r.at[tidx4.at[p * CPP + j]], rbufs[j], sem)
               for j in range(CPP)]

        @plsc.parallel_loop(0, RPP // 16, unroll=2)
        def grp(g):
            cols = pl.ds(prow0 + g * 16, 16)
            lv = li[cols]
            vv = vi[cols]
            b_sl = (prow0 + g * 16 + lane) * SL
            toks = [plsc.load_gather(si, [b_sl + l]) for l in range(SL)]
            row_off = (g * 16 + lane) * OD
            for t in range(D // 2):
                dvec = (lane + t) & (D // 2 - 1)
                wl = plsc.bitcast(
                    plsc.load_gather(loctab, [lv + dvec * LOCV]), jnp.bfloat16)
                a, b = plsc.unpack(wl, format=plsc.PackFormat.INTERLEAVED)
                plsc.store_scatter(tile, [row_off + D + dvec], a)
                plsc.store_scatter(tile, [row_off + D + 16 + dvec], b)
                wv = plsc.bitcast(
                    plsc.load_gather(levtab, [vv + dvec * LEVV]), jnp.bfloat16)
                a, b = plsc.unpack(wv, format=plsc.PackFormat.INTERLEAVED)
                plsc.store_scatter(tile, [row_off + 2 * D + dvec], a)
                plsc.store_scatter(tile, [row_off + 2 * D + 16 + dvec], b)
                dsk = dvec * SKV
                vals = [plsc.bitcast(plsc.load_gather(sktab, [toks[l] + dsk]),
                                     jnp.bfloat16)
                        for l in range(SL)]
                while len(vals) > 1:
                    vals = [a + b for a, b in zip(vals[::2], vals[1::2])] + (
                        [vals[-1]] if len(vals) % 2 else [])
                a, b = plsc.unpack(vals[0], format=plsc.PackFormat.INTERLEAVED)
                plsc.store_scatter(tile, [row_off + 3 * D + dvec], a * inv_len)
                plsc.store_scatter(tile, [row_off + 3 * D + 16 + dvec],
                                   b * inv_len)

        for j in range(CPP):
            cps[j].wait()
            rbj = rbufs[j]
            crow0 = prow0 + j * CHUNK

            @plsc.parallel_loop(0, CHUNK // 16, unroll=2)
            def tgrp(k):
                r_vec = k * 16 + lane
                idxv = tio[pl.ds(crow0 + k * 16, 16)]
                is_oov = idxv == TITLE_OOV
                sub = (idxv & 3) * D
                row_off = (j * CHUNK + k * 16 + lane) * OD
                for t in range(D):
                    dvec = (lane + t) & (D - 1)
                    val = plsc.load_gather(rbj, [r_vec, sub + dvec])
                    val = jnp.where(is_oov, plsc.load_gather(oov, [dvec]), val)
                    plsc.store_scatter(tile, [row_off + dvec], val)

        pltpu.sync_copy(tile, out.at[pl.ds((base + prow0) * OD, RPP * OD)])
        return carry

    lax.fori_loop(0, NPASS, one_pass, 0)


@jax.jit
def kernel(item1, location_item1, level_item1, skill_text_item1,
           title_table, location_table, level_table, skill_table):
    mesh = plsc.VectorSubcoreMesh(core_axis_name="c", subcore_axis_name="s",
                                  num_cores=NC, num_subcores=NS)
    f32 = jnp.float32
    title_r = title_table[:4 * TROWS].reshape(TROWS, 4 * D)
    oov_row = title_table[TITLE_OOV]

    def pack_pairs(tab):
        bf = tab.astype(jnp.bfloat16)
        lo = lax.bitcast_convert_type(bf[:, :16], jnp.uint16).astype(jnp.uint32)
        hi = lax.bitcast_convert_type(bf[:, 16:], jnp.uint16).astype(jnp.uint32)
        return (lo | (hi << 16)).astype(jnp.int32).T.reshape(-1)

    loctab_t = pack_pairs(location_table)
    levtab_t = pack_pairs(level_table)
    sktab_t = pack_pairs(skill_table)
    run = pl.kernel(
        _sc_body,
        out_type=jax.ShapeDtypeStruct((B * OD,), f32),
        mesh=mesh,
        compiler_params=pltpu.CompilerParams(needs_layout_passes=False),
        scratch_types=[
            pltpu.VMEM((BPW,), jnp.int32),
            pltpu.VMEM((NCHUNK, CHUNK), jnp.int32),
            pltpu.VMEM((BPW,), jnp.int32),
            pltpu.VMEM((BPW,), jnp.int32),
            pltpu.VMEM((BPW * SL,), jnp.int32),
            pltpu.VMEM((D,), f32),
            pltpu.VMEM((D // 2 * LOCV,), jnp.int32),
            pltpu.VMEM((D // 2 * LEVV,), jnp.int32),
            pltpu.VMEM((D // 2 * SKV,), jnp.int32),
            pltpu.VMEM((RPP * OD,), f32),
            pltpu.VMEM((CHUNK, 4 * D), f32),
            pltpu.VMEM((CHUNK, 4 * D), f32),
            pltpu.SemaphoreType.DMA,
        ],
    )
    flat = run(item1, location_item1, level_item1,
               skill_text_item1.reshape(-1),
               title_r, oov_row, loctab_t, levtab_t, sktab_t)
    return flat.reshape(B, OD)
